# baseline (speedup 1.0000x reference)
import functools

import jax
import jax.numpy as jnp
from jax import lax
from jax.experimental import pallas as pl
from jax.experimental.pallas import tpu as pltpu
from jax.experimental.pallas import tpu_sc as plsc

N_NODES = 10000
D = 128
E = 320000
BLK = 128
NB = E // BLK
NC = 2
NS = 16
NW = NC * NS
EB = 80
N_PAD = 10240
ROWS_PER_TILE = N_PAD // NS
DEG_W = 128

_sc_mesh = plsc.VectorSubcoreMesh(core_axis_name="c", subcore_axis_name="s")


@functools.partial(
    pl.kernel,
    out_type=jax.ShapeDtypeStruct((NC, N_PAD, DEG_W), jnp.float32),
    mesh=_sc_mesh,
    scratch_types=[
        pltpu.VMEM((BLK,), jnp.int32),
        pltpu.VMEM((BLK, DEG_W), jnp.float32),
        pltpu.VMEM_SHARED((N_PAD, DEG_W), jnp.float32),
    ],
)
def _deg_kernel(dst_hbm, ones_hbm, zeros_hbm, out_hbm, idx_d, ones_v, accum):
    cid = lax.axis_index("c")
    sid = lax.axis_index("s")
    wid = sid * NC + cid
    pltpu.sync_copy(ones_hbm, ones_v)
    pltpu.sync_copy(zeros_hbm, accum.at[pl.ds(sid * ROWS_PER_TILE, ROWS_PER_TILE)])
    plsc.subcore_barrier()

    def body(j, carry):
        off = (wid + j * NW) * BLK
        pltpu.sync_copy(dst_hbm.at[pl.ds(off, BLK)], idx_d)
        pltpu.sync_copy(ones_v, accum.at[idx_d], add=True)
        return carry

    lax.fori_loop(0, EB, body, 0)
    plsc.subcore_barrier()
    pltpu.sync_copy(
        accum.at[pl.ds(sid * ROWS_PER_TILE, ROWS_PER_TILE)],
        out_hbm.at[cid, pl.ds(sid * ROWS_PER_TILE, ROWS_PER_TILE)],
    )


@functools.partial(
    pl.kernel,
    out_type=jax.ShapeDtypeStruct((NC, N_PAD, D), jnp.float32),
    mesh=_sc_mesh,
    scratch_types=[
        pltpu.VMEM((BLK,), jnp.int32),
        pltpu.VMEM((BLK,), jnp.int32),
        pltpu.VMEM((BLK, D), jnp.float32),
        pltpu.VMEM_SHARED((N_PAD, D), jnp.float32),
        pltpu.SemaphoreType.DMA,
    ],
)
def _scatter_kernel(g_hbm, src_hbm, dst_hbm, zeros_hbm, out_hbm,
                    idx_s, idx_d, rows, accum, sem):
    cid = lax.axis_index("c")
    sid = lax.axis_index("s")
    wid = sid * NC + cid
    pltpu.sync_copy(zeros_hbm, accum.at[pl.ds(sid * ROWS_PER_TILE, ROWS_PER_TILE)])
    plsc.subcore_barrier()

    def body(j, carry):
        off = (wid + j * NW) * BLK
        pltpu.sync_copy(src_hbm.at[pl.ds(off, BLK)], idx_s)
        pltpu.sync_copy(dst_hbm.at[pl.ds(off, BLK)], idx_d)
        pltpu.async_copy(g_hbm.at[idx_s], rows, sem).wait()
        pltpu.sync_copy(rows, accum.at[idx_d], add=True)
        return carry

    lax.fori_loop(0, EB, body, 0)
    plsc.subcore_barrier()
    pltpu.sync_copy(
        accum.at[pl.ds(sid * ROWS_PER_TILE, ROWS_PER_TILE)],
        out_hbm.at[cid, pl.ds(sid * ROWS_PER_TILE, ROWS_PER_TILE)],
    )


_ROWS_BLK = 2000
_GRID = N_NODES // _ROWS_BLK


def _dinv_block(degp_ref):
    d = degp_ref[0, :, 0:1] + degp_ref[1, :, 0:1] + 1.0
    return lax.rsqrt(d)


def _tc0_body(x_ref, degp_ref, wt_ref, b_ref, out0_ref, g1_ref):
    x = x_ref[...]
    dinv = _dinv_block(degp_ref)
    out0_ref[...] = jnp.dot(x, wt_ref[...], preferred_element_type=jnp.float32) + b_ref[...]
    g1_ref[...] = x * dinv


def _tc_mid_body(sp_ref, g_ref, degp_ref, wt_ref, b_ref, out_ref, gnext_ref):
    dinv = _dinv_block(degp_ref)
    h = (sp_ref[0] + sp_ref[1] + g_ref[...]) * dinv
    out_ref[...] = jnp.dot(h, wt_ref[...], preferred_element_type=jnp.float32) + b_ref[...]
    gnext_ref[...] = h * dinv


def _tc_last_body(sp_ref, g_ref, degp_ref, wt_ref, b_ref, out_ref):
    dinv = _dinv_block(degp_ref)
    h = (sp_ref[0] + sp_ref[1] + g_ref[...]) * dinv
    out_ref[...] = jnp.dot(h, wt_ref[...], preferred_element_type=jnp.float32) + b_ref[...]


_x_spec = pl.BlockSpec((_ROWS_BLK, D), lambda i: (i, 0))
_sp_spec = pl.BlockSpec((NC, _ROWS_BLK, D), lambda i: (0, i, 0))
_degp_spec = pl.BlockSpec((NC, _ROWS_BLK, DEG_W), lambda i: (0, i, 0))
_w_spec = pl.BlockSpec((D, D), lambda i: (0, 0))
_b_spec = pl.BlockSpec((1, D), lambda i: (0, 0))
_out_shape = jax.ShapeDtypeStruct((N_NODES, D), jnp.float32)

_tc0 = pl.pallas_call(
    _tc0_body,
    grid=(_GRID,),
    in_specs=[_x_spec, _degp_spec, _w_spec, _b_spec],
    out_specs=[_x_spec, _x_spec],
    out_shape=[_out_shape, _out_shape],
)

_tc_mid = pl.pallas_call(
    _tc_mid_body,
    grid=(_GRID,),
    in_specs=[_sp_spec, _x_spec, _degp_spec, _w_spec, _b_spec],
    out_specs=[_x_spec, _x_spec],
    out_shape=[_out_shape, _out_shape],
)

_tc_last = pl.pallas_call(
    _tc_last_body,
    grid=(_GRID,),
    in_specs=[_sp_spec, _x_spec, _degp_spec, _w_spec, _b_spec],
    out_specs=_x_spec,
    out_shape=_out_shape,
)


_FAKE_SRC = jnp.zeros((NW * EB * BLK - E,), jnp.int32)
_FAKE_DST = N_NODES + jnp.arange(NW * EB * BLK - E, dtype=jnp.int32) % (N_PAD - N_NODES)


def kernel(x, edge_index, W0, b0, W1, b1, W2, b2):
    ei = edge_index.astype(jnp.int32)
    src = jnp.concatenate([ei[0], _FAKE_SRC])
    dst = jnp.concatenate([ei[1], _FAKE_DST])
    ones_deg = jnp.ones((BLK, DEG_W), jnp.float32)
    zeros_deg = jnp.zeros((ROWS_PER_TILE, DEG_W), jnp.float32)
    zeros_rows = jnp.zeros((ROWS_PER_TILE, D), jnp.float32)

    degp = _deg_kernel(dst, ones_deg, zeros_deg)
    out0, g1 = _tc0(x, degp, W0.T, b0.reshape(1, D))
    sp1 = _scatter_kernel(g1, src, dst, zeros_rows)
    out1, g2 = _tc_mid(sp1, g1, degp, W1.T, b1.reshape(1, D))
    sp2 = _scatter_kernel(g2, src, dst, zeros_rows)
    out2 = _tc_last(sp2, g2, degp, W2.T, b2.reshape(1, D))
    return jnp.concatenate([out0, out1, out2], axis=-1)


# speedup vs baseline: 1.7951x; 1.7951x over previous
import functools

import jax
import jax.numpy as jnp
from jax import lax
from jax.experimental import pallas as pl
from jax.experimental.pallas import tpu as pltpu
from jax.experimental.pallas import tpu_sc as plsc

N_NODES = 10000
D = 128
E = 320000
BLK = 128
NB = E // BLK
NC = 2
NS = 16
NW = NC * NS
EB = 80
N_PAD = 10240
ROWS_PER_TILE = N_PAD // NS
DEG_W = 128

_sc_mesh = plsc.VectorSubcoreMesh(core_axis_name="c", subcore_axis_name="s")


@functools.partial(
    pl.kernel,
    out_type=jax.ShapeDtypeStruct((NC, N_PAD, DEG_W), jnp.float32),
    mesh=_sc_mesh,
    scratch_types=[
        pltpu.VMEM((BLK,), jnp.int32),
        pltpu.VMEM((BLK, DEG_W), jnp.float32),
        pltpu.VMEM_SHARED((N_PAD, DEG_W), jnp.float32),
    ],
)
def _deg_kernel(dst_hbm, ones_hbm, zeros_hbm, out_hbm, idx_d, ones_v, accum):
    cid = lax.axis_index("c")
    sid = lax.axis_index("s")
    wid = sid * NC + cid
    pltpu.sync_copy(ones_hbm, ones_v)
    pltpu.sync_copy(zeros_hbm, accum.at[pl.ds(sid * ROWS_PER_TILE, ROWS_PER_TILE)])
    plsc.subcore_barrier()

    def body(j, carry):
        off = (wid + j * NW) * BLK
        pltpu.sync_copy(dst_hbm.at[pl.ds(off, BLK)], idx_d)
        pltpu.sync_copy(ones_v, accum.at[idx_d], add=True)
        return carry

    lax.fori_loop(0, EB, body, 0)
    plsc.subcore_barrier()
    pltpu.sync_copy(
        accum.at[pl.ds(sid * ROWS_PER_TILE, ROWS_PER_TILE)],
        out_hbm.at[cid, pl.ds(sid * ROWS_PER_TILE, ROWS_PER_TILE)],
    )


@functools.partial(
    pl.kernel,
    out_type=jax.ShapeDtypeStruct((NC, N_PAD, D), jnp.float32),
    mesh=_sc_mesh,
    scratch_types=[
        pltpu.VMEM((BLK,), jnp.int32),
        pltpu.VMEM((BLK,), jnp.int32),
        pltpu.VMEM((BLK, D), jnp.float32),
        pltpu.VMEM_SHARED((N_PAD, D), jnp.float32),
        pltpu.SemaphoreType.DMA,
    ],
)
def _scatter_kernel(g_hbm, src_hbm, dst_hbm, zeros_hbm, out_hbm,
                    idx_s, idx_d, rows, accum, sem):
    cid = lax.axis_index("c")
    sid = lax.axis_index("s")
    wid = sid * NC + cid
    pltpu.sync_copy(zeros_hbm, accum.at[pl.ds(sid * ROWS_PER_TILE, ROWS_PER_TILE)])
    plsc.subcore_barrier()

    def body(j, carry):
        off = (wid + j * NW) * BLK
        pltpu.sync_copy(src_hbm.at[pl.ds(off, BLK)], idx_s)
        pltpu.sync_copy(dst_hbm.at[pl.ds(off, BLK)], idx_d)
        pltpu.async_copy(g_hbm.at[idx_s], rows, sem).wait()
        pltpu.sync_copy(rows, accum.at[idx_d], add=True)
        return carry

    lax.fori_loop(0, EB, body, 0)
    plsc.subcore_barrier()
    pltpu.sync_copy(
        accum.at[pl.ds(sid * ROWS_PER_TILE, ROWS_PER_TILE)],
        out_hbm.at[cid, pl.ds(sid * ROWS_PER_TILE, ROWS_PER_TILE)],
    )


_ROWS_BLK = 2000
_GRID = N_NODES // _ROWS_BLK


def _dinv_block(degp_ref):
    d = degp_ref[0, :, 0:1] + degp_ref[1, :, 0:1] + 1.0
    return lax.rsqrt(d)


def _tc0_body(x_ref, degp_ref, wt_ref, b_ref, out0_ref, g1_ref):
    x = x_ref[...]
    dinv = _dinv_block(degp_ref)
    out0_ref[...] = jnp.dot(x, wt_ref[...], preferred_element_type=jnp.float32) + b_ref[...]
    g1_ref[...] = x * dinv


def _tc_mid_body(sp_ref, g_ref, degp_ref, wt_ref, b_ref, out_ref, gnext_ref):
    dinv = _dinv_block(degp_ref)
    h = (sp_ref[0] + sp_ref[1] + g_ref[...]) * dinv
    out_ref[...] = jnp.dot(h, wt_ref[...], preferred_element_type=jnp.float32) + b_ref[...]
    gnext_ref[...] = h * dinv


def _tc_last_body(sp_ref, g_ref, degp_ref, wt_ref, b_ref, out_ref):
    dinv = _dinv_block(degp_ref)
    h = (sp_ref[0] + sp_ref[1] + g_ref[...]) * dinv
    out_ref[...] = jnp.dot(h, wt_ref[...], preferred_element_type=jnp.float32) + b_ref[...]


_x_spec = pl.BlockSpec((_ROWS_BLK, D), lambda i: (i, 0))
_sp_spec = pl.BlockSpec((NC, _ROWS_BLK, D), lambda i: (0, i, 0))
_degp_spec = pl.BlockSpec((NC, _ROWS_BLK, DEG_W), lambda i: (0, i, 0))
_w_spec = pl.BlockSpec((D, D), lambda i: (0, 0))
_b_spec = pl.BlockSpec((1, D), lambda i: (0, 0))
_out_shape = jax.ShapeDtypeStruct((N_NODES, D), jnp.float32)

_tc0 = pl.pallas_call(
    _tc0_body,
    grid=(_GRID,),
    in_specs=[_x_spec, _degp_spec, _w_spec, _b_spec],
    out_specs=[_x_spec, _x_spec],
    out_shape=[_out_shape, _out_shape],
)

_tc_mid = pl.pallas_call(
    _tc_mid_body,
    grid=(_GRID,),
    in_specs=[_sp_spec, _x_spec, _degp_spec, _w_spec, _b_spec],
    out_specs=[_x_spec, _x_spec],
    out_shape=[_out_shape, _out_shape],
)

_tc_last = pl.pallas_call(
    _tc_last_body,
    grid=(_GRID,),
    in_specs=[_sp_spec, _x_spec, _degp_spec, _w_spec, _b_spec],
    out_specs=_x_spec,
    out_shape=_out_shape,
)


_FAKE_SRC = jnp.arange(NW * EB * BLK - E, dtype=jnp.int32) % N_NODES
_FAKE_DST = N_NODES + jnp.arange(NW * EB * BLK - E, dtype=jnp.int32) % (N_PAD - N_NODES)


def kernel(x, edge_index, W0, b0, W1, b1, W2, b2):
    ei = edge_index.astype(jnp.int32)
    src = jnp.concatenate([ei[0], _FAKE_SRC])
    dst = jnp.concatenate([ei[1], _FAKE_DST])
    ones_deg = jnp.ones((BLK, DEG_W), jnp.float32)
    zeros_deg = jnp.zeros((ROWS_PER_TILE, DEG_W), jnp.float32)
    zeros_rows = jnp.zeros((ROWS_PER_TILE, D), jnp.float32)

    degp = _deg_kernel(dst, ones_deg, zeros_deg)
    out0, g1 = _tc0(x, degp, W0.T, b0.reshape(1, D))
    sp1 = _scatter_kernel(g1, src, dst, zeros_rows)
    out1, g2 = _tc_mid(sp1, g1, degp, W1.T, b1.reshape(1, D))
    sp2 = _scatter_kernel(g2, src, dst, zeros_rows)
    out2 = _tc_last(sp2, g2, degp, W2.T, b2.reshape(1, D))
    return jnp.concatenate([out0, out1, out2], axis=-1)


# speedup vs baseline: 2.5155x; 1.4014x over previous
import functools

import jax
import jax.numpy as jnp
from jax import lax
from jax.experimental import pallas as pl
from jax.experimental.pallas import tpu as pltpu
from jax.experimental.pallas import tpu_sc as plsc

N_NODES = 10000
D = 128
E = 320000
BLK = 128
NB = E // BLK
NC = 2
NS = 16
NW = NC * NS
EB = 80
E_PAD = NW * EB * BLK
NBUF = 2
N_PAD = 10240
ROWS_PER_TILE = N_PAD // NS
DEG_W = 128

_sc_mesh = plsc.VectorSubcoreMesh(core_axis_name="c", subcore_axis_name="s")


@functools.partial(
    pl.kernel,
    out_type=jax.ShapeDtypeStruct((NC, N_PAD, DEG_W), jnp.float32),
    mesh=_sc_mesh,
    scratch_types=[
        pltpu.VMEM((EB, BLK), jnp.int32),
        pltpu.VMEM((BLK, DEG_W), jnp.float32),
        pltpu.VMEM_SHARED((N_PAD, DEG_W), jnp.float32),
        pltpu.SemaphoreType.DMA((NBUF,)),
    ],
)
def _deg_kernel(dstb_hbm, ones_hbm, zeros_hbm, out_hbm, idx_d, ones_v, accum, ssem):
    cid = lax.axis_index("c")
    sid = lax.axis_index("s")
    wid = sid * NC + cid
    pltpu.sync_copy(ones_hbm, ones_v)
    pltpu.sync_copy(dstb_hbm.at[pl.ds(wid * EB, EB)], idx_d)
    pltpu.sync_copy(zeros_hbm, accum.at[pl.ds(sid * ROWS_PER_TILE, ROWS_PER_TILE)])
    plsc.subcore_barrier()

    def outer(o, carry):
        base = o * NBUF
        pend = [
            pltpu.async_copy(ones_v, accum.at[idx_d.at[base + b]], ssem.at[b], add=True)
            for b in range(NBUF)
        ]
        for d in pend:
            d.wait()
        return carry

    lax.fori_loop(0, EB // NBUF, outer, 0)
    plsc.subcore_barrier()
    pltpu.sync_copy(
        accum.at[pl.ds(sid * ROWS_PER_TILE, ROWS_PER_TILE)],
        out_hbm.at[cid, pl.ds(sid * ROWS_PER_TILE, ROWS_PER_TILE)],
    )


@functools.partial(
    pl.kernel,
    out_type=jax.ShapeDtypeStruct((NC, N_PAD, D), jnp.float32),
    mesh=_sc_mesh,
    scratch_types=[
        pltpu.VMEM((BLK,), jnp.int32),
        pltpu.VMEM((BLK,), jnp.int32),
        pltpu.VMEM((BLK,), jnp.int32),
        pltpu.VMEM((BLK,), jnp.int32),
        pltpu.VMEM((BLK, D), jnp.float32),
        pltpu.VMEM((BLK, D), jnp.float32),
        pltpu.VMEM_SHARED((N_PAD, D), jnp.float32),
        pltpu.SemaphoreType.DMA((4,)),
        pltpu.SemaphoreType.DMA((2,)),
        pltpu.SemaphoreType.DMA((2,)),
    ],
)
def _scatter_kernel(g_hbm, src1_hbm, dst1_hbm, zeros_hbm, out_hbm,
                    isA, idA, isB, idB, rowsA, rowsB, accum, isem, gsem, ssem):
    cid = lax.axis_index("c")
    sid = lax.axis_index("s")
    wid = sid * NC + cid
    pltpu.sync_copy(zeros_hbm, accum.at[pl.ds(sid * ROWS_PER_TILE, ROWS_PER_TILE)])
    plsc.subcore_barrier()

    def outer(o, carry):
        e0 = ((2 * o) * NW + wid) * BLK
        e1 = e0 + NW * BLK
        i0 = pltpu.async_copy(src1_hbm.at[pl.ds(e0, BLK)], isA, isem.at[0])
        i1 = pltpu.async_copy(dst1_hbm.at[pl.ds(e0, BLK)], idA, isem.at[1])
        i2 = pltpu.async_copy(src1_hbm.at[pl.ds(e1, BLK)], isB, isem.at[2])
        i3 = pltpu.async_copy(dst1_hbm.at[pl.ds(e1, BLK)], idB, isem.at[3])
        i0.wait()
        gA = pltpu.async_copy(g_hbm.at[isA], rowsA, gsem.at[0])
        i2.wait()
        gB = pltpu.async_copy(g_hbm.at[isB], rowsB, gsem.at[1])
        gA.wait()
        i1.wait()
        sA = pltpu.async_copy(rowsA, accum.at[idA], ssem.at[0], add=True)
        gB.wait()
        i3.wait()
        sB = pltpu.async_copy(rowsB, accum.at[idB], ssem.at[1], add=True)
        sA.wait()
        sB.wait()
        return carry

    lax.fori_loop(0, EB // 2, outer, 0)
    plsc.subcore_barrier()
    pltpu.sync_copy(
        accum.at[pl.ds(sid * ROWS_PER_TILE, ROWS_PER_TILE)],
        out_hbm.at[cid, pl.ds(sid * ROWS_PER_TILE, ROWS_PER_TILE)],
    )


_ROWS_BLK = 2000
_GRID = N_NODES // _ROWS_BLK


def _dinv_block(degp_ref):
    d = degp_ref[0, :, 0:1] + degp_ref[1, :, 0:1] + 1.0
    return lax.rsqrt(d)


def _tc0_body(x_ref, degp_ref, wt_ref, b_ref, out0_ref, g1_ref):
    x = x_ref[...]
    dinv = _dinv_block(degp_ref)
    out0_ref[...] = jnp.dot(x, wt_ref[...], preferred_element_type=jnp.float32) + b_ref[...]
    g1_ref[...] = x * dinv


def _tc_mid_body(sp_ref, g_ref, degp_ref, wt_ref, b_ref, out_ref, gnext_ref):
    dinv = _dinv_block(degp_ref)
    h = (sp_ref[0] + sp_ref[1] + g_ref[...]) * dinv
    out_ref[...] = jnp.dot(h, wt_ref[...], preferred_element_type=jnp.float32) + b_ref[...]
    gnext_ref[...] = h * dinv


def _tc_last_body(sp_ref, g_ref, degp_ref, wt_ref, b_ref, out_ref):
    dinv = _dinv_block(degp_ref)
    h = (sp_ref[0] + sp_ref[1] + g_ref[...]) * dinv
    out_ref[...] = jnp.dot(h, wt_ref[...], preferred_element_type=jnp.float32) + b_ref[...]


_x_spec = pl.BlockSpec((_ROWS_BLK, D), lambda i: (i, 0))
_sp_spec = pl.BlockSpec((NC, _ROWS_BLK, D), lambda i: (0, i, 0))
_degp_spec = pl.BlockSpec((NC, _ROWS_BLK, DEG_W), lambda i: (0, i, 0))
_w_spec = pl.BlockSpec((D, D), lambda i: (0, 0))
_b_spec = pl.BlockSpec((1, D), lambda i: (0, 0))
_out_shape = jax.ShapeDtypeStruct((N_NODES, D), jnp.float32)

_tc0 = pl.pallas_call(
    _tc0_body,
    grid=(_GRID,),
    in_specs=[_x_spec, _degp_spec, _w_spec, _b_spec],
    out_specs=[_x_spec, _x_spec],
    out_shape=[_out_shape, _out_shape],
)

_tc_mid = pl.pallas_call(
    _tc_mid_body,
    grid=(_GRID,),
    in_specs=[_sp_spec, _x_spec, _degp_spec, _w_spec, _b_spec],
    out_specs=[_x_spec, _x_spec],
    out_shape=[_out_shape, _out_shape],
)

_tc_last = pl.pallas_call(
    _tc_last_body,
    grid=(_GRID,),
    in_specs=[_sp_spec, _x_spec, _degp_spec, _w_spec, _b_spec],
    out_specs=_x_spec,
    out_shape=_out_shape,
)


_FAKE_SRC = jnp.arange(NW * EB * BLK - E, dtype=jnp.int32) % N_NODES
_FAKE_DST = N_NODES + jnp.arange(NW * EB * BLK - E, dtype=jnp.int32) % (N_PAD - N_NODES)


def kernel(x, edge_index, W0, b0, W1, b1, W2, b2):
    ei = edge_index.astype(jnp.int32)
    src1 = jnp.concatenate([ei[0], _FAKE_SRC])
    dst1 = jnp.concatenate([ei[1], _FAKE_DST])
    dstb = dst1.reshape(NW * EB, BLK)
    ones_deg = jnp.ones((BLK, DEG_W), jnp.float32)
    zeros_deg = jnp.zeros((ROWS_PER_TILE, DEG_W), jnp.float32)
    zeros_rows = jnp.zeros((ROWS_PER_TILE, D), jnp.float32)

    degp = _deg_kernel(dstb, ones_deg, zeros_deg)
    out0, g1 = _tc0(x, degp, W0.T, b0.reshape(1, D))
    sp1 = _scatter_kernel(g1, src1, dst1, zeros_rows)
    out1, g2 = _tc_mid(sp1, g1, degp, W1.T, b1.reshape(1, D))
    sp2 = _scatter_kernel(g2, src1, dst1, zeros_rows)
    out2 = _tc_last(sp2, g2, degp, W2.T, b2.reshape(1, D))
    return jnp.concatenate([out0, out1, out2], axis=-1)
